# baseline (speedup 1.0000x reference)
import functools
import jax
import jax.numpy as jnp
from jax import lax
from jax.experimental import pallas as pl
from jax.experimental.pallas import tpu as pltpu
from jax.experimental.pallas import tpu_sc as plsc

_CB = 8


def kernel(x, pe):
    B, L, D = x.shape
    info = plsc.get_sparse_core_info()
    NC, NS, LN = info.num_cores, info.num_subcores, info.num_lanes
    NW = NC * NS
    rows_w = B // NW
    nchunk = rows_w // _CB

    @functools.partial(
        pl.kernel,
        mesh=plsc.VectorSubcoreMesh(core_axis_name="c", subcore_axis_name="s"),
        out_type=jax.ShapeDtypeStruct((B, L, D), jnp.float32),
        scratch_types=[
            pltpu.VMEM((pe.shape[0], D), jnp.float32),
            pltpu.VMEM((_CB, L, D), jnp.float32),
        ],
    )
    def _k(x_hbm, pe_hbm, out_hbm, pe_v, buf):
        wid = lax.axis_index("s") * NC + lax.axis_index("c")
        base = wid * rows_w
        pltpu.sync_copy(pe_hbm, pe_v)

        def chunk_body(ci, _):
            start = base + ci * _CB
            pltpu.sync_copy(x_hbm.at[pl.ds(start, _CB)], buf)

            def row_body(r, _):
                for b in range(_CB):
                    for l in range(D // LN):
                        sl = pl.ds(l * LN, LN)
                        buf[b, r, sl] = buf[b, r, sl] + pe_v[r, sl]
                return 0

            lax.fori_loop(0, L, row_body, 0)
            pltpu.sync_copy(buf, out_hbm.at[pl.ds(start, _CB)])
            return 0

        lax.fori_loop(0, nchunk, chunk_body, 0)

    return _k(x, pe)


# speedup vs baseline: 1.3017x; 1.3017x over previous
import functools
import jax
import jax.numpy as jnp
from jax import lax
from jax.experimental import pallas as pl
from jax.experimental.pallas import tpu as pltpu
from jax.experimental.pallas import tpu_sc as plsc

_CB = 4
_NBUF = 2


def kernel(x, pe):
    B, L, D = x.shape
    info = plsc.get_sparse_core_info()
    NC, NS, LN = info.num_cores, info.num_subcores, info.num_lanes
    NW = NC * NS
    rows_w = B // NW
    nchunk = rows_w // _CB

    @functools.partial(
        pl.kernel,
        mesh=plsc.VectorSubcoreMesh(core_axis_name="c", subcore_axis_name="s"),
        out_type=jax.ShapeDtypeStruct((B, L, D), jnp.float32),
        scratch_types=[
            pltpu.VMEM((pe.shape[0], D), jnp.float32),
            pltpu.VMEM((_CB, L, D), jnp.float32),
            pltpu.VMEM((_CB, L, D), jnp.float32),
            pltpu.VMEM((_CB, L, D), jnp.float32),
            pltpu.VMEM((_CB, L, D), jnp.float32),
            pltpu.SemaphoreType.DMA,
            pltpu.SemaphoreType.DMA,
            pltpu.SemaphoreType.DMA,
            pltpu.SemaphoreType.DMA,
        ],
    )
    def _k(x_hbm, pe_hbm, out_hbm, pe_v, i0, i1, o0, o1, si0, si1, so0, so1):
        wid = lax.axis_index("s") * NC + lax.axis_index("c")
        base = wid * rows_w
        pltpu.sync_copy(pe_hbm, pe_v)
        ibufs, obufs = (i0, i1), (o0, o1)
        sin, sout = (si0, si1), (so0, so1)

        def in_cp(i, b):
            return pltpu.make_async_copy(
                x_hbm.at[pl.ds(base + i * _CB, _CB)], ibufs[b], sin[b])

        def out_cp(i, b):
            return pltpu.make_async_copy(
                obufs[b], out_hbm.at[pl.ds(base + i * _CB, _CB)], sout[b])

        in_cp(0, 0).start()
        in_cp(1, 1).start()

        def outer(g, _):
            for b in range(_NBUF):
                i = g * _NBUF + b
                in_cp(i, b).wait()

                @pl.when(i >= _NBUF)
                def _():
                    out_cp(i - _NBUF, b).wait()

                ib, ob = ibufs[b], obufs[b]

                def row_body(r, _):
                    for bb in range(_CB):
                        for l in range(D // LN):
                            sl = pl.ds(l * LN, LN)
                            ob[bb, r, sl] = ib[bb, r, sl] + pe_v[r, sl]
                    return 0

                lax.fori_loop(0, L, row_body, 0)
                out_cp(i, b).start()

                @pl.when(i + _NBUF < nchunk)
                def _():
                    in_cp(i + _NBUF, b).start()
            return 0

        lax.fori_loop(0, nchunk // _NBUF, outer, 0)
        out_cp(nchunk - 2, 0).wait()
        out_cp(nchunk - 1, 1).wait()

    return _k(x, pe)
